# Initial kernel scaffold; baseline (speedup 1.0000x reference)
import jax
import jax.numpy as jnp
from jax.experimental import pallas as pl


def kernel(features, edge_index, W1, b1, W2, b2, Wfc, bfc):
    raise NotImplementedError("write your pallas kernel here")


# speedup vs baseline: 14.7392x; 14.7392x over previous
import functools

import jax
import jax.numpy as jnp
from jax import lax
from jax.experimental import pallas as pl
from jax.experimental.pallas import tpu as pltpu
from jax.experimental.pallas import tpu_sc as plsc

_N = 10000
_E = 320000
_DIN = 128
_H1 = 512
_H2 = 256
_EMB = 128

_NC = 2
_NS = 16
_NW = _NC * _NS
_C = 128
_NCHUNK = _E // _C
_RPT = _N // _NS

_mesh = plsc.VectorSubcoreMesh(core_axis_name="c", subcore_axis_name="s")
_sc_params = pltpu.CompilerParams(use_tc_tiling_on_sc=False)


def _deg_body(dst_hbm, zeros_hbm, ones_hbm, out_hbm, idx_v, ones_v, acc_sh, sem):
    c = lax.axis_index("c")
    s = lax.axis_index("s")
    w = s * _NC + c
    pltpu.sync_copy(zeros_hbm.at[pl.ds(s * _RPT, _RPT)],
                    acc_sh.at[pl.ds(s * _RPT, _RPT)])
    pltpu.sync_copy(ones_hbm, ones_v)
    plsc.subcore_barrier()
    nch = (_NCHUNK - w + _NW - 1) // _NW

    def chunk(k, carry):
        j = w + k * _NW
        pltpu.sync_copy(dst_hbm.at[pl.ds(j * _C, _C)], idx_v.at[0])
        pltpu.sync_copy(ones_v, acc_sh.at[idx_v.at[0]], add=True)
        return carry

    lax.fori_loop(0, nch, chunk, 0)
    plsc.subcore_barrier()
    pltpu.sync_copy(acc_sh.at[pl.ds(s * _RPT, _RPT)],
                    out_hbm.at[c, pl.ds(s * _RPT, _RPT)])


_deg = pl.kernel(
    _deg_body,
    out_type=jax.ShapeDtypeStruct((_NC, _N, 16), jnp.float32),
    mesh=_mesh,
    scratch_types=[
        pltpu.VMEM((1, _C), jnp.int32),
        pltpu.VMEM((_C, 16), jnp.float32),
        pltpu.VMEM_SHARED((_N, 16), jnp.float32),
        pltpu.SemaphoreType.DMA,
    ],
    compiler_params=_sc_params,
)


def _agg_body(x_hbm, src2_hbm, dst_hbm, out_hbm, idxs_v, idxd_v, rows_v, acc_sh,
              sem, *, dh):
    c = lax.axis_index("c")
    s = lax.axis_index("s")
    pltpu.sync_copy(x_hbm.at[pl.ds(c * _N + s * _RPT, _RPT)],
                    acc_sh.at[pl.ds(s * _RPT, _RPT)])
    plsc.subcore_barrier()
    nch = (_NCHUNK - s + _NS - 1) // _NS

    def chunk(k, carry):
        base = (s + k * _NS) * _C
        pltpu.sync_copy(src2_hbm.at[c, pl.ds(base, _C)], idxs_v.at[0])
        pltpu.sync_copy(dst_hbm.at[pl.ds(base, _C)], idxd_v.at[0])
        pltpu.async_copy(x_hbm.at[idxs_v.at[0]], rows_v, sem).wait()
        pltpu.sync_copy(rows_v, acc_sh.at[idxd_v.at[0]], add=True)
        return carry

    lax.fori_loop(0, nch, chunk, 0)
    plsc.subcore_barrier()
    pltpu.sync_copy(acc_sh.at[pl.ds(s * _RPT, _RPT)],
                    out_hbm.at[c, pl.ds(s * _RPT, _RPT)])


@functools.cache
def _agg(dh):
    return pl.kernel(
        functools.partial(_agg_body, dh=dh),
        out_type=jax.ShapeDtypeStruct((_NC, _N, dh), jnp.float32),
        mesh=_mesh,
        scratch_types=[
            pltpu.VMEM((1, _C), jnp.int32),
            pltpu.VMEM((1, _C), jnp.int32),
            pltpu.VMEM((_C, dh), jnp.float32),
            pltpu.VMEM_SHARED((_N, dh), jnp.float32),
            pltpu.SemaphoreType.DMA,
        ],
        compiler_params=_sc_params,
    )


_BLK = 1000
_GRID = _N // _BLK


def _prep_body(feat_ref, degp_ref, x2_ref, dis_ref):
    deg = 1.0 + degp_ref[0, :, 0:1] + degp_ref[1, :, 0:1]
    dis = lax.rsqrt(deg)
    dis_ref[...] = dis
    x2_ref[0] = feat_ref[:, : _DIN // 2] * dis
    x2_ref[1] = feat_ref[:, _DIN // 2:] * dis


def _prep(features, degp):
    return pl.pallas_call(
        _prep_body,
        grid=(_GRID,),
        in_specs=[
            pl.BlockSpec((_BLK, _DIN), lambda i: (i, 0)),
            pl.BlockSpec((_NC, _BLK, 16), lambda i: (0, i, 0)),
        ],
        out_specs=[
            pl.BlockSpec((_NC, _BLK, _DIN // 2), lambda i: (0, i, 0)),
            pl.BlockSpec((_BLK, 1), lambda i: (i, 0)),
        ],
        out_shape=[
            jax.ShapeDtypeStruct((_NC, _N, _DIN // 2), jnp.float32),
            jax.ShapeDtypeStruct((_N, 1), jnp.float32),
        ],
    )(features, degp)


def _mm_t(a, w):
    return lax.dot_general(a, w, (((1,), (1,)), ((), ())),
                           preferred_element_type=jnp.float32)


def _mid_body(t1_ref, dis_ref, W1_ref, b1_ref, W2_ref, x3_ref):
    dis = dis_ref[...]
    aggx = jnp.concatenate([t1_ref[0], t1_ref[1]], axis=1) * dis
    u = jnp.maximum(_mm_t(aggx, W1_ref[...]) + b1_ref[...], 0.0)
    v = _mm_t(u, W2_ref[...]) * dis
    x3_ref[0] = v[:, : _H2 // 2]
    x3_ref[1] = v[:, _H2 // 2:]


def _mid(t1, dis, W1, b1, W2):
    return pl.pallas_call(
        _mid_body,
        grid=(_GRID,),
        in_specs=[
            pl.BlockSpec((_NC, _BLK, _DIN // 2), lambda i: (0, i, 0)),
            pl.BlockSpec((_BLK, 1), lambda i: (i, 0)),
            pl.BlockSpec((_H1, _DIN), lambda i: (0, 0)),
            pl.BlockSpec((1, _H1), lambda i: (0, 0)),
            pl.BlockSpec((_H2, _H1), lambda i: (0, 0)),
        ],
        out_specs=pl.BlockSpec((_NC, _BLK, _H2 // 2), lambda i: (0, i, 0)),
        out_shape=jax.ShapeDtypeStruct((_NC, _N, _H2 // 2), jnp.float32),
    )(t1, dis, W1, b1, W2)


def _fin_body(t2_ref, dis_ref, b2_ref, Wfc_ref, bfc_ref, out_ref, gmax_scr):
    i = pl.program_id(0)
    w = jnp.concatenate([t2_ref[0], t2_ref[1]], axis=1) * dis_ref[...] + b2_ref[...]
    m = jnp.max(w, axis=0, keepdims=True)

    @pl.when(i == 0)
    def _():
        gmax_scr[...] = m

    @pl.when(i > 0)
    def _():
        gmax_scr[...] = jnp.maximum(gmax_scr[...], m)

    @pl.when(i == pl.num_programs(0) - 1)
    def _():
        out_ref[...] = _mm_t(gmax_scr[...], Wfc_ref[...]) + bfc_ref[...]


def _fin(t2, dis, b2, Wfc, bfc):
    return pl.pallas_call(
        _fin_body,
        grid=(_GRID,),
        in_specs=[
            pl.BlockSpec((_NC, _BLK, _H2 // 2), lambda i: (0, i, 0)),
            pl.BlockSpec((_BLK, 1), lambda i: (i, 0)),
            pl.BlockSpec((1, _H2), lambda i: (0, 0)),
            pl.BlockSpec((_EMB, _H2), lambda i: (0, 0)),
            pl.BlockSpec((1, _EMB), lambda i: (0, 0)),
        ],
        out_specs=pl.BlockSpec((1, _EMB), lambda i: (0, 0)),
        out_shape=jax.ShapeDtypeStruct((1, _EMB), jnp.float32),
        scratch_shapes=[pltpu.VMEM((1, _H2), jnp.float32)],
    )(t2, dis, b2, Wfc, bfc)


def kernel(features, edge_index, W1, b1, W2, b2, Wfc, bfc):
    src = edge_index[0].astype(jnp.int32)
    dst = edge_index[1].astype(jnp.int32)
    src2 = jnp.stack([src, src + _N])
    zeros_n16 = jnp.zeros((_N, 16), jnp.float32)
    ones_c16 = jnp.ones((_C, 16), jnp.float32)

    degp = _deg(dst, zeros_n16, ones_c16)
    x2, dis = _prep(features.astype(jnp.float32), degp)
    t1 = _agg(_DIN // 2)(x2.reshape(2 * _N, _DIN // 2), src2, dst)
    x3 = _mid(t1, dis, W1, b1.reshape(1, _H1), W2)
    t2 = _agg(_H2 // 2)(x3.reshape(2 * _N, _H2 // 2), src2, dst)
    out = _fin(t2, dis, b2.reshape(1, _H2), Wfc, bfc.reshape(1, _EMB))
    return out.reshape(_EMB)


# speedup vs baseline: 29.2586x; 1.9851x over previous
import functools

import jax
import jax.numpy as jnp
from jax import lax
from jax.experimental import pallas as pl
from jax.experimental.pallas import tpu as pltpu
from jax.experimental.pallas import tpu_sc as plsc

_N = 10000
_E = 320000
_DIN = 128
_H1 = 512
_H2 = 256
_EMB = 128

_NC = 2
_NS = 16
_NW = _NC * _NS
_C = 125
_NROW = _E // _C
_NCH = _NROW // _NS
_DCH = _NROW // _NW
_NBUF = 4
_RPT = _N // _NS

_mesh = plsc.VectorSubcoreMesh(core_axis_name="c", subcore_axis_name="s")
_sc_params = pltpu.CompilerParams(use_tc_tiling_on_sc=False)


def _deg_body(dst_hbm, zeros_hbm, ones_hbm, out_hbm, idx_v, ones_v, acc_sh,
              *sems):
    c = lax.axis_index("c")
    s = lax.axis_index("s")
    w = s * _NC + c
    pltpu.sync_copy(zeros_hbm.at[pl.ds(s * _RPT, _RPT)],
                    acc_sh.at[pl.ds(s * _RPT, _RPT)])
    pltpu.sync_copy(dst_hbm.at[pl.ds(w * _DCH, _DCH)], idx_v)
    pltpu.sync_copy(ones_hbm, ones_v)
    plsc.subcore_barrier()

    def group(g, carry):
        for b in range(_NBUF):
            k = g * _NBUF + b

            @pl.when(g > 0)
            def _():
                pltpu.make_async_copy(ones_v, acc_sh.at[idx_v.at[k - _NBUF]],
                                      sems[b]).wait()

            pltpu.async_copy(ones_v, acc_sh.at[idx_v.at[k]], sems[b], add=True)
        return carry

    lax.fori_loop(0, _DCH // _NBUF, group, 0)
    for b in range(_NBUF):
        k = _DCH - _NBUF + b
        pltpu.make_async_copy(ones_v, acc_sh.at[idx_v.at[k]], sems[b]).wait()
    plsc.subcore_barrier()
    pltpu.sync_copy(acc_sh.at[pl.ds(s * _RPT, _RPT)],
                    out_hbm.at[c, pl.ds(s * _RPT, _RPT)])


_deg = pl.kernel(
    _deg_body,
    out_type=jax.ShapeDtypeStruct((_NC, _N, 16), jnp.float32),
    mesh=_mesh,
    scratch_types=[
        pltpu.VMEM((_DCH, _C), jnp.int32),
        pltpu.VMEM((_C, 16), jnp.float32),
        pltpu.VMEM_SHARED((_N, 16), jnp.float32),
    ] + [pltpu.SemaphoreType.DMA] * _NBUF,
    compiler_params=_sc_params,
)


@functools.cache
def _agg(dh, nbuf):
    def body(x_hbm, sd_hbm, out_hbm, idx_v, rows_v, acc_sh, *sems):
        isem = sems[:nbuf]
        gsem = sems[nbuf:2 * nbuf]
        ssem = sems[2 * nbuf:]
        c = lax.axis_index("c")
        s = lax.axis_index("s")
        row0 = s * _NCH
        for b in range(nbuf):
            pltpu.async_copy(sd_hbm.at[c, row0 + b], idx_v.at[b], isem[b])
        pltpu.sync_copy(x_hbm.at[pl.ds(c * _N + s * _RPT, _RPT)],
                        acc_sh.at[pl.ds(s * _RPT, _RPT)])
        for b in range(nbuf):
            pltpu.make_async_copy(sd_hbm.at[c, row0 + b], idx_v.at[b],
                                  isem[b]).wait()
            pltpu.async_copy(x_hbm.at[idx_v.at[b, 0]], rows_v.at[b], gsem[b])
        plsc.subcore_barrier()

        def group(g, carry):
            k0 = g * nbuf
            for b in range(nbuf):
                pltpu.make_async_copy(x_hbm.at[idx_v.at[b, 0]], rows_v.at[b],
                                      gsem[b]).wait()
                pltpu.async_copy(rows_v.at[b], acc_sh.at[idx_v.at[b, 1]],
                                 ssem[b], add=True)
            for b in range(nbuf):
                k = k0 + b
                pltpu.make_async_copy(rows_v.at[b], acc_sh.at[idx_v.at[b, 1]],
                                      ssem[b]).wait()

                @pl.when(k + nbuf < _NCH)
                def _():
                    kn = row0 + k + nbuf
                    pltpu.async_copy(sd_hbm.at[c, kn], idx_v.at[b], isem[b])
                    pltpu.make_async_copy(sd_hbm.at[c, kn], idx_v.at[b],
                                          isem[b]).wait()
                    pltpu.async_copy(x_hbm.at[idx_v.at[b, 0]], rows_v.at[b],
                                     gsem[b])
            return carry

        lax.fori_loop(0, _NCH // nbuf, group, 0)
        plsc.subcore_barrier()
        pltpu.sync_copy(acc_sh.at[pl.ds(s * _RPT, _RPT)],
                        out_hbm.at[c, pl.ds(s * _RPT, _RPT)])

    return pl.kernel(
        body,
        out_type=jax.ShapeDtypeStruct((_NC, _N, dh), jnp.float32),
        mesh=_mesh,
        scratch_types=[
            pltpu.VMEM((nbuf, 2, _C), jnp.int32),
            pltpu.VMEM((nbuf, _C, dh), jnp.float32),
            pltpu.VMEM_SHARED((_N, dh), jnp.float32),
        ] + [pltpu.SemaphoreType.DMA] * (3 * nbuf),
        compiler_params=_sc_params,
    )


_BLK = 1000
_GRID = _N // _BLK


def _prep_body(feat_ref, degp_ref, x2_ref, dis_ref):
    deg = 1.0 + degp_ref[0, :, 0:1] + degp_ref[1, :, 0:1]
    dis = lax.rsqrt(deg)
    dis_ref[...] = dis
    x2_ref[0] = feat_ref[:, : _DIN // 2] * dis
    x2_ref[1] = feat_ref[:, _DIN // 2:] * dis


def _prep(features, degp):
    return pl.pallas_call(
        _prep_body,
        grid=(_GRID,),
        in_specs=[
            pl.BlockSpec((_BLK, _DIN), lambda i: (i, 0)),
            pl.BlockSpec((_NC, _BLK, 16), lambda i: (0, i, 0)),
        ],
        out_specs=[
            pl.BlockSpec((_NC, _BLK, _DIN // 2), lambda i: (0, i, 0)),
            pl.BlockSpec((_BLK, 1), lambda i: (i, 0)),
        ],
        out_shape=[
            jax.ShapeDtypeStruct((_NC, _N, _DIN // 2), jnp.float32),
            jax.ShapeDtypeStruct((_N, 1), jnp.float32),
        ],
    )(features, degp)


def _mm_t(a, w):
    return lax.dot_general(a, w, (((1,), (1,)), ((), ())),
                           preferred_element_type=jnp.float32)


def _mid_body(t1_ref, dis_ref, W1_ref, b1_ref, W2_ref, x3_ref):
    dis = dis_ref[...]
    aggx = jnp.concatenate([t1_ref[0], t1_ref[1]], axis=1) * dis
    u = jnp.maximum(_mm_t(aggx, W1_ref[...]) + b1_ref[...], 0.0)
    v = _mm_t(u, W2_ref[...]) * dis
    x3_ref[0] = v[:, : _H2 // 2]
    x3_ref[1] = v[:, _H2 // 2:]


def _mid(t1, dis, W1, b1, W2):
    return pl.pallas_call(
        _mid_body,
        grid=(_GRID,),
        in_specs=[
            pl.BlockSpec((_NC, _BLK, _DIN // 2), lambda i: (0, i, 0)),
            pl.BlockSpec((_BLK, 1), lambda i: (i, 0)),
            pl.BlockSpec((_H1, _DIN), lambda i: (0, 0)),
            pl.BlockSpec((1, _H1), lambda i: (0, 0)),
            pl.BlockSpec((_H2, _H1), lambda i: (0, 0)),
        ],
        out_specs=pl.BlockSpec((_NC, _BLK, _H2 // 2), lambda i: (0, i, 0)),
        out_shape=jax.ShapeDtypeStruct((_NC, _N, _H2 // 2), jnp.float32),
    )(t1, dis, W1, b1, W2)


def _fin_body(t2_ref, dis_ref, b2_ref, Wfc_ref, bfc_ref, out_ref, gmax_scr):
    i = pl.program_id(0)
    w = jnp.concatenate([t2_ref[0], t2_ref[1]], axis=1) * dis_ref[...] + b2_ref[...]
    m = jnp.max(w, axis=0, keepdims=True)

    @pl.when(i == 0)
    def _():
        gmax_scr[...] = m

    @pl.when(i > 0)
    def _():
        gmax_scr[...] = jnp.maximum(gmax_scr[...], m)

    @pl.when(i == pl.num_programs(0) - 1)
    def _():
        out_ref[...] = _mm_t(gmax_scr[...], Wfc_ref[...]) + bfc_ref[...]


def _fin(t2, dis, b2, Wfc, bfc):
    return pl.pallas_call(
        _fin_body,
        grid=(_GRID,),
        in_specs=[
            pl.BlockSpec((_NC, _BLK, _H2 // 2), lambda i: (0, i, 0)),
            pl.BlockSpec((_BLK, 1), lambda i: (i, 0)),
            pl.BlockSpec((1, _H2), lambda i: (0, 0)),
            pl.BlockSpec((_EMB, _H2), lambda i: (0, 0)),
            pl.BlockSpec((1, _EMB), lambda i: (0, 0)),
        ],
        out_specs=pl.BlockSpec((1, _EMB), lambda i: (0, 0)),
        out_shape=jax.ShapeDtypeStruct((1, _EMB), jnp.float32),
        scratch_shapes=[pltpu.VMEM((1, _H2), jnp.float32)],
    )(t2, dis, b2, Wfc, bfc)


def kernel(features, edge_index, W1, b1, W2, b2, Wfc, bfc):
    src = edge_index[0].astype(jnp.int32)
    dst = edge_index[1].astype(jnp.int32)
    srcr = src.reshape(1, _NROW, 1, _C)
    dstr = dst.reshape(1, _NROW, 1, _C)
    off = jnp.arange(_NC, dtype=jnp.int32).reshape(_NC, 1, 1, 1) * _N
    sd = jnp.concatenate(
        [jnp.broadcast_to(srcr + off, (_NC, _NROW, 1, _C)),
         jnp.broadcast_to(dstr, (_NC, _NROW, 1, _C))], axis=2)
    dst3 = dst.reshape(_NROW, _C)
    zeros_n16 = jnp.zeros((_N, 16), jnp.float32)
    ones_c16 = jnp.ones((_C, 16), jnp.float32)

    degp = _deg(dst3, zeros_n16, ones_c16)
    x2, dis = _prep(features.astype(jnp.float32), degp)
    t1 = _agg(_DIN // 2, 4)(x2.reshape(2 * _N, _DIN // 2), sd)
    x3 = _mid(t1, dis, W1, b1.reshape(1, _H1), W2)
    t2 = _agg(_H2 // 2, 2)(x3.reshape(2 * _N, _H2 // 2), sd)
    out = _fin(t2, dis, b2.reshape(1, _H2), Wfc, bfc.reshape(1, _EMB))
    return out.reshape(_EMB)


# speedup vs baseline: 29.7295x; 1.0161x over previous
import functools

import jax
import jax.numpy as jnp
from jax import lax
from jax.experimental import pallas as pl
from jax.experimental.pallas import tpu as pltpu
from jax.experimental.pallas import tpu_sc as plsc

_N = 10000
_E = 320000
_DIN = 128
_H1 = 512
_H2 = 256
_EMB = 128

_NC = 2
_NS = 16
_NW = _NC * _NS
_C = 125
_NROW = _E // _C
_NCH = _NROW // _NS
_DCH = _NROW // _NW
_NBUF = 4
_RPT = _N // _NS

_mesh = plsc.VectorSubcoreMesh(core_axis_name="c", subcore_axis_name="s")
_sc_params = pltpu.CompilerParams(use_tc_tiling_on_sc=False)


def _deg_body(dst_hbm, zeros_hbm, ones_hbm, out_hbm, idx_v, ones_v, acc_sh,
              *sems):
    c = lax.axis_index("c")
    s = lax.axis_index("s")
    w = s * _NC + c
    pltpu.sync_copy(zeros_hbm.at[pl.ds(s * _RPT, _RPT)],
                    acc_sh.at[pl.ds(s * _RPT, _RPT)])
    pltpu.sync_copy(dst_hbm.at[pl.ds(w * _DCH, _DCH)], idx_v)
    pltpu.sync_copy(ones_hbm, ones_v)
    plsc.subcore_barrier()

    def group(g, carry):
        for b in range(_NBUF):
            k = g * _NBUF + b

            @pl.when(g > 0)
            def _():
                pltpu.make_async_copy(ones_v, acc_sh.at[idx_v.at[k - _NBUF]],
                                      sems[b]).wait()

            pltpu.async_copy(ones_v, acc_sh.at[idx_v.at[k]], sems[b], add=True)
        return carry

    lax.fori_loop(0, _DCH // _NBUF, group, 0)
    for b in range(_NBUF):
        k = _DCH - _NBUF + b
        pltpu.make_async_copy(ones_v, acc_sh.at[idx_v.at[k]], sems[b]).wait()
    plsc.subcore_barrier()
    pltpu.sync_copy(acc_sh.at[pl.ds(s * _RPT, _RPT)],
                    out_hbm.at[c, pl.ds(s * _RPT, _RPT)])


_deg = pl.kernel(
    _deg_body,
    out_type=jax.ShapeDtypeStruct((_NC, _N, 16), jnp.float32),
    mesh=_mesh,
    scratch_types=[
        pltpu.VMEM((_DCH, _C), jnp.int32),
        pltpu.VMEM((_C, 16), jnp.float32),
        pltpu.VMEM_SHARED((_N, 16), jnp.float32),
    ] + [pltpu.SemaphoreType.DMA] * _NBUF,
    compiler_params=_sc_params,
)


@functools.cache
def _agg(dh, nbuf):
    ni = 2 * nbuf

    def body(x_hbm, sd_hbm, out_hbm, idx_v, rows_v, acc_sh, *sems):
        isem = sems[:ni]
        gsem = sems[ni:ni + nbuf]
        ssem = sems[ni + nbuf:]
        c = lax.axis_index("c")
        s = lax.axis_index("s")
        row0 = s * _NCH
        for q in range(ni):
            pltpu.async_copy(sd_hbm.at[c, row0 + q], idx_v.at[q], isem[q])
        pltpu.sync_copy(x_hbm.at[pl.ds(c * _N + s * _RPT, _RPT)],
                        acc_sh.at[pl.ds(s * _RPT, _RPT)])
        for b in range(nbuf):
            pltpu.make_async_copy(sd_hbm.at[c, row0 + b], idx_v.at[b],
                                  isem[b]).wait()
            pltpu.async_copy(x_hbm.at[idx_v.at[b, 0]], rows_v.at[b], gsem[b])
        plsc.subcore_barrier()

        def group(g, carry):
            for p in range(2):
                k0 = g * ni + p * nbuf
                for b in range(nbuf):
                    j = p * nbuf + b
                    pltpu.make_async_copy(x_hbm.at[idx_v.at[j, 0]],
                                          rows_v.at[b], gsem[b]).wait()
                    pltpu.async_copy(rows_v.at[b], acc_sh.at[idx_v.at[j, 1]],
                                     ssem[b], add=True)
                for b in range(nbuf):
                    j = p * nbuf + b
                    jn = (j + nbuf) % ni
                    k = k0 + b
                    pltpu.make_async_copy(rows_v.at[b],
                                          acc_sh.at[idx_v.at[j, 1]],
                                          ssem[b]).wait()

                    @pl.when(k + nbuf < _NCH)
                    def _():
                        pltpu.make_async_copy(sd_hbm.at[c, row0 + k + nbuf],
                                              idx_v.at[jn], isem[jn]).wait()
                        pltpu.async_copy(x_hbm.at[idx_v.at[jn, 0]],
                                         rows_v.at[b], gsem[b])

                    @pl.when(k + ni < _NCH)
                    def _():
                        pltpu.async_copy(sd_hbm.at[c, row0 + k + ni],
                                         idx_v.at[j], isem[j])
            return carry

        lax.fori_loop(0, _NCH // ni, group, 0)
        plsc.subcore_barrier()
        pltpu.sync_copy(acc_sh.at[pl.ds(s * _RPT, _RPT)],
                        out_hbm.at[c, pl.ds(s * _RPT, _RPT)])

    return pl.kernel(
        body,
        out_type=jax.ShapeDtypeStruct((_NC, _N, dh), jnp.float32),
        mesh=_mesh,
        scratch_types=[
            pltpu.VMEM((ni, 2, _C), jnp.int32),
            pltpu.VMEM((nbuf, _C, dh), jnp.float32),
            pltpu.VMEM_SHARED((_N, dh), jnp.float32),
        ] + [pltpu.SemaphoreType.DMA] * (ni + 2 * nbuf),
        compiler_params=_sc_params,
    )


_BLK = 1000
_GRID = _N // _BLK


def _prep_body(feat_ref, degp_ref, x2_ref, dis_ref):
    deg = 1.0 + degp_ref[0, :, 0:1] + degp_ref[1, :, 0:1]
    dis = lax.rsqrt(deg)
    dis_ref[...] = dis
    x2_ref[0] = feat_ref[:, : _DIN // 2] * dis
    x2_ref[1] = feat_ref[:, _DIN // 2:] * dis


def _prep(features, degp):
    return pl.pallas_call(
        _prep_body,
        grid=(_GRID,),
        in_specs=[
            pl.BlockSpec((_BLK, _DIN), lambda i: (i, 0)),
            pl.BlockSpec((_NC, _BLK, 16), lambda i: (0, i, 0)),
        ],
        out_specs=[
            pl.BlockSpec((_NC, _BLK, _DIN // 2), lambda i: (0, i, 0)),
            pl.BlockSpec((_BLK, 1), lambda i: (i, 0)),
        ],
        out_shape=[
            jax.ShapeDtypeStruct((_NC, _N, _DIN // 2), jnp.float32),
            jax.ShapeDtypeStruct((_N, 1), jnp.float32),
        ],
    )(features, degp)


def _mm_t(a, w):
    return lax.dot_general(a, w, (((1,), (1,)), ((), ())),
                           preferred_element_type=jnp.float32)


def _mid_body(t1_ref, dis_ref, W1_ref, b1_ref, W2_ref, x3_ref):
    dis = dis_ref[...]
    aggx = jnp.concatenate([t1_ref[0], t1_ref[1]], axis=1) * dis
    u = jnp.maximum(_mm_t(aggx, W1_ref[...]) + b1_ref[...], 0.0)
    v = _mm_t(u, W2_ref[...]) * dis
    x3_ref[0] = v[:, : _H2 // 2]
    x3_ref[1] = v[:, _H2 // 2:]


def _mid(t1, dis, W1, b1, W2):
    return pl.pallas_call(
        _mid_body,
        grid=(_GRID,),
        in_specs=[
            pl.BlockSpec((_NC, _BLK, _DIN // 2), lambda i: (0, i, 0)),
            pl.BlockSpec((_BLK, 1), lambda i: (i, 0)),
            pl.BlockSpec((_H1, _DIN), lambda i: (0, 0)),
            pl.BlockSpec((1, _H1), lambda i: (0, 0)),
            pl.BlockSpec((_H2, _H1), lambda i: (0, 0)),
        ],
        out_specs=pl.BlockSpec((_NC, _BLK, _H2 // 2), lambda i: (0, i, 0)),
        out_shape=jax.ShapeDtypeStruct((_NC, _N, _H2 // 2), jnp.float32),
    )(t1, dis, W1, b1, W2)


def _fin_body(t2_ref, dis_ref, b2_ref, Wfc_ref, bfc_ref, out_ref, gmax_scr):
    i = pl.program_id(0)
    w = jnp.concatenate([t2_ref[0], t2_ref[1]], axis=1) * dis_ref[...] + b2_ref[...]
    m = jnp.max(w, axis=0, keepdims=True)

    @pl.when(i == 0)
    def _():
        gmax_scr[...] = m

    @pl.when(i > 0)
    def _():
        gmax_scr[...] = jnp.maximum(gmax_scr[...], m)

    @pl.when(i == pl.num_programs(0) - 1)
    def _():
        out_ref[...] = _mm_t(gmax_scr[...], Wfc_ref[...]) + bfc_ref[...]


def _fin(t2, dis, b2, Wfc, bfc):
    return pl.pallas_call(
        _fin_body,
        grid=(_GRID,),
        in_specs=[
            pl.BlockSpec((_NC, _BLK, _H2 // 2), lambda i: (0, i, 0)),
            pl.BlockSpec((_BLK, 1), lambda i: (i, 0)),
            pl.BlockSpec((1, _H2), lambda i: (0, 0)),
            pl.BlockSpec((_EMB, _H2), lambda i: (0, 0)),
            pl.BlockSpec((1, _EMB), lambda i: (0, 0)),
        ],
        out_specs=pl.BlockSpec((1, _EMB), lambda i: (0, 0)),
        out_shape=jax.ShapeDtypeStruct((1, _EMB), jnp.float32),
        scratch_shapes=[pltpu.VMEM((1, _H2), jnp.float32)],
    )(t2, dis, b2, Wfc, bfc)


def kernel(features, edge_index, W1, b1, W2, b2, Wfc, bfc):
    src = edge_index[0].astype(jnp.int32)
    dst = edge_index[1].astype(jnp.int32)
    srcr = src.reshape(1, _NROW, 1, _C)
    dstr = dst.reshape(1, _NROW, 1, _C)
    off = jnp.arange(_NC, dtype=jnp.int32).reshape(_NC, 1, 1, 1) * _N
    sd = jnp.concatenate(
        [jnp.broadcast_to(srcr + off, (_NC, _NROW, 1, _C)),
         jnp.broadcast_to(dstr, (_NC, _NROW, 1, _C))], axis=2)
    dst3 = dst.reshape(_NROW, _C)
    zeros_n16 = jnp.zeros((_N, 16), jnp.float32)
    ones_c16 = jnp.ones((_C, 16), jnp.float32)

    degp = _deg(dst3, zeros_n16, ones_c16)
    x2, dis = _prep(features.astype(jnp.float32), degp)
    t1 = _agg(_DIN // 2, 4)(x2.reshape(2 * _N, _DIN // 2), sd)
    x3 = _mid(t1, dis, W1, b1.reshape(1, _H1), W2)
    t2 = _agg(_H2 // 2, 2)(x3.reshape(2 * _N, _H2 // 2), sd)
    out = _fin(t2, dis, b2.reshape(1, _H2), Wfc, bfc.reshape(1, _EMB))
    return out.reshape(_EMB)


# speedup vs baseline: 29.9583x; 1.0077x over previous
import functools

import jax
import jax.numpy as jnp
from jax import lax
from jax.experimental import pallas as pl
from jax.experimental.pallas import tpu as pltpu
from jax.experimental.pallas import tpu_sc as plsc

_N = 10000
_E = 320000
_DIN = 128
_H1 = 512
_H2 = 256
_EMB = 128

_NC = 2
_NS = 16
_NW = _NC * _NS
_C = 125
_NROW = _E // _C
_NCH = _NROW // _NS
_DCH = _NROW // _NW
_NBUF = 4
_RPT = _N // _NS

_mesh = plsc.VectorSubcoreMesh(core_axis_name="c", subcore_axis_name="s")
_sc_params = pltpu.CompilerParams(use_tc_tiling_on_sc=False)


def _deg_body(dst_hbm, zeros_hbm, ones_hbm, out_hbm, idx_v, ones_v, acc_sh,
              *sems):
    c = lax.axis_index("c")
    s = lax.axis_index("s")
    w = s * _NC + c
    pltpu.sync_copy(zeros_hbm.at[pl.ds(s * _RPT, _RPT)],
                    acc_sh.at[pl.ds(s * _RPT, _RPT)])
    pltpu.sync_copy(dst_hbm.at[pl.ds(w * _DCH, _DCH)], idx_v)
    pltpu.sync_copy(ones_hbm, ones_v)
    plsc.subcore_barrier()

    def group(g, carry):
        for b in range(_NBUF):
            k = g * _NBUF + b

            @pl.when(g > 0)
            def _():
                pltpu.make_async_copy(ones_v, acc_sh.at[idx_v.at[k - _NBUF]],
                                      sems[b]).wait()

            pltpu.async_copy(ones_v, acc_sh.at[idx_v.at[k]], sems[b], add=True)
        return carry

    lax.fori_loop(0, _DCH // _NBUF, group, 0)
    for b in range(_NBUF):
        k = _DCH - _NBUF + b
        pltpu.make_async_copy(ones_v, acc_sh.at[idx_v.at[k]], sems[b]).wait()
    plsc.subcore_barrier()
    pltpu.sync_copy(acc_sh.at[pl.ds(s * _RPT, _RPT)],
                    out_hbm.at[c, pl.ds(s * _RPT, _RPT)])


_deg = pl.kernel(
    _deg_body,
    out_type=jax.ShapeDtypeStruct((_NC, _N, 16), jnp.float32),
    mesh=_mesh,
    scratch_types=[
        pltpu.VMEM((_DCH, _C), jnp.int32),
        pltpu.VMEM((_C, 16), jnp.float32),
        pltpu.VMEM_SHARED((_N, 16), jnp.float32),
    ] + [pltpu.SemaphoreType.DMA] * _NBUF,
    compiler_params=_sc_params,
)


@functools.cache
def _agg(dh, nbuf):
    ni = 2 * nbuf

    def body(x_hbm, sd_hbm, out_hbm, idx_v, rows_v, acc_sh, *sems):
        isem = sems[:ni]
        gsem = sems[ni:ni + nbuf]
        ssem = sems[ni + nbuf:]
        c = lax.axis_index("c")
        s = lax.axis_index("s")
        row0 = s * _NCH
        for q in range(ni):
            pltpu.async_copy(sd_hbm.at[c, row0 + q], idx_v.at[q], isem[q])
        pltpu.sync_copy(x_hbm.at[pl.ds(c * _N + s * _RPT, _RPT)],
                        acc_sh.at[pl.ds(s * _RPT, _RPT)])
        for b in range(nbuf):
            pltpu.make_async_copy(sd_hbm.at[c, row0 + b], idx_v.at[b],
                                  isem[b]).wait()
            pltpu.async_copy(x_hbm.at[idx_v.at[b, 0]], rows_v.at[b], gsem[b])
        plsc.subcore_barrier()

        def group(g, carry):
            for p in range(2):
                k0 = g * ni + p * nbuf
                for b in range(nbuf):
                    j = p * nbuf + b
                    pltpu.make_async_copy(x_hbm.at[idx_v.at[j, 0]],
                                          rows_v.at[b], gsem[b]).wait()
                    pltpu.async_copy(rows_v.at[b], acc_sh.at[idx_v.at[j, 1]],
                                     ssem[b], add=True)
                for b in range(nbuf):
                    j = p * nbuf + b
                    jn = (j + nbuf) % ni
                    k = k0 + b
                    pltpu.make_async_copy(rows_v.at[b],
                                          acc_sh.at[idx_v.at[j, 1]],
                                          ssem[b]).wait()

                    @pl.when(k + nbuf < _NCH)
                    def _():
                        pltpu.make_async_copy(sd_hbm.at[c, row0 + k + nbuf],
                                              idx_v.at[jn], isem[jn]).wait()
                        pltpu.async_copy(x_hbm.at[idx_v.at[jn, 0]],
                                         rows_v.at[b], gsem[b])

                    @pl.when(k + ni < _NCH)
                    def _():
                        pltpu.async_copy(sd_hbm.at[c, row0 + k + ni],
                                         idx_v.at[j], isem[j])
            return carry

        lax.fori_loop(0, _NCH // ni, group, 0)
        plsc.subcore_barrier()
        pltpu.sync_copy(acc_sh.at[pl.ds(s * _RPT, _RPT)],
                        out_hbm.at[c, pl.ds(s * _RPT, _RPT)])

    return pl.kernel(
        body,
        out_type=jax.ShapeDtypeStruct((_NC, _N, dh), jnp.float32),
        mesh=_mesh,
        scratch_types=[
            pltpu.VMEM((ni, 2, _C), jnp.int32),
            pltpu.VMEM((nbuf, _C, dh), jnp.float32),
            pltpu.VMEM_SHARED((_N, dh), jnp.float32),
        ] + [pltpu.SemaphoreType.DMA] * (ni + 2 * nbuf),
        compiler_params=_sc_params,
    )


_BLK = 1000
_GRID = _N // _BLK


def _prep_body(feat_ref, degp_ref, x2_ref, dis_ref):
    deg = 1.0 + degp_ref[0, :, 0:1] + degp_ref[1, :, 0:1]
    dis = lax.rsqrt(deg)
    dis_ref[...] = dis
    x2_ref[0] = feat_ref[:, : _DIN // 2] * dis
    x2_ref[1] = feat_ref[:, _DIN // 2:] * dis


def _prep(features, degp):
    return pl.pallas_call(
        _prep_body,
        grid=(_GRID,),
        in_specs=[
            pl.BlockSpec((_BLK, _DIN), lambda i: (i, 0)),
            pl.BlockSpec((_NC, _BLK, 16), lambda i: (0, i, 0)),
        ],
        out_specs=[
            pl.BlockSpec((_NC, _BLK, _DIN // 2), lambda i: (0, i, 0)),
            pl.BlockSpec((_BLK, 1), lambda i: (i, 0)),
        ],
        out_shape=[
            jax.ShapeDtypeStruct((_NC, _N, _DIN // 2), jnp.float32),
            jax.ShapeDtypeStruct((_N, 1), jnp.float32),
        ],
    )(features, degp)


def _mm_t(a, w):
    return lax.dot_general(a, w, (((1,), (1,)), ((), ())),
                           preferred_element_type=jnp.float32)


def _mid_body(t1_ref, dis_ref, W1_ref, b1_ref, W2_ref, x3_ref):
    dis = dis_ref[...]
    aggx = jnp.concatenate([t1_ref[0], t1_ref[1]], axis=1) * dis
    u = jnp.maximum(_mm_t(aggx, W1_ref[...]) + b1_ref[...], 0.0)
    v = _mm_t(u, W2_ref[...]) * dis
    x3_ref[0] = v[:, : _H2 // 2]
    x3_ref[1] = v[:, _H2 // 2:]


def _mid(t1, dis, W1, b1, W2):
    return pl.pallas_call(
        _mid_body,
        grid=(_GRID,),
        in_specs=[
            pl.BlockSpec((_NC, _BLK, _DIN // 2), lambda i: (0, i, 0)),
            pl.BlockSpec((_BLK, 1), lambda i: (i, 0)),
            pl.BlockSpec((_H1, _DIN), lambda i: (0, 0)),
            pl.BlockSpec((1, _H1), lambda i: (0, 0)),
            pl.BlockSpec((_H2, _H1), lambda i: (0, 0)),
        ],
        out_specs=pl.BlockSpec((_NC, _BLK, _H2 // 2), lambda i: (0, i, 0)),
        out_shape=jax.ShapeDtypeStruct((_NC, _N, _H2 // 2), jnp.float32),
    )(t1, dis, W1, b1, W2)


def _fin_body(t2_ref, dis_ref, b2_ref, Wfc_ref, bfc_ref, out_ref, gmax_scr):
    i = pl.program_id(0)
    w = jnp.concatenate([t2_ref[0], t2_ref[1]], axis=1) * dis_ref[...] + b2_ref[...]
    m = jnp.max(w, axis=0, keepdims=True)

    @pl.when(i == 0)
    def _():
        gmax_scr[...] = m

    @pl.when(i > 0)
    def _():
        gmax_scr[...] = jnp.maximum(gmax_scr[...], m)

    @pl.when(i == pl.num_programs(0) - 1)
    def _():
        out_ref[...] = _mm_t(gmax_scr[...], Wfc_ref[...]) + bfc_ref[...]


def _fin(t2, dis, b2, Wfc, bfc):
    return pl.pallas_call(
        _fin_body,
        grid=(_GRID,),
        in_specs=[
            pl.BlockSpec((_NC, _BLK, _H2 // 2), lambda i: (0, i, 0)),
            pl.BlockSpec((_BLK, 1), lambda i: (i, 0)),
            pl.BlockSpec((1, _H2), lambda i: (0, 0)),
            pl.BlockSpec((_EMB, _H2), lambda i: (0, 0)),
            pl.BlockSpec((1, _EMB), lambda i: (0, 0)),
        ],
        out_specs=pl.BlockSpec((1, _EMB), lambda i: (0, 0)),
        out_shape=jax.ShapeDtypeStruct((1, _EMB), jnp.float32),
        scratch_shapes=[pltpu.VMEM((1, _H2), jnp.float32)],
    )(t2, dis, b2, Wfc, bfc)


def kernel(features, edge_index, W1, b1, W2, b2, Wfc, bfc):
    src = edge_index[0].astype(jnp.int32)
    dst = edge_index[1].astype(jnp.int32)
    srcr = src.reshape(1, _NROW, 1, _C)
    dstr = dst.reshape(1, _NROW, 1, _C)
    off = jnp.arange(_NC, dtype=jnp.int32).reshape(_NC, 1, 1, 1) * _N
    sd = jnp.concatenate(
        [jnp.broadcast_to(srcr + off, (_NC, _NROW, 1, _C)),
         jnp.broadcast_to(dstr, (_NC, _NROW, 1, _C))], axis=2)
    dst3 = dst.reshape(_NROW, _C)
    zeros_n16 = jnp.zeros((_N, 16), jnp.float32)
    ones_c16 = jnp.ones((_C, 16), jnp.float32)

    degp = _deg(dst3, zeros_n16, ones_c16)
    x2, dis = _prep(features.astype(jnp.float32), degp)
    t1 = _agg(_DIN // 2, 5)(x2.reshape(2 * _N, _DIN // 2), sd)
    x3 = _mid(t1, dis, W1, b1.reshape(1, _H1), W2)
    t2 = _agg(_H2 // 2, 2)(x3.reshape(2 * _N, _H2 // 2), sd)
    out = _fin(t2, dis, b2.reshape(1, _H2), Wfc, bfc.reshape(1, _EMB))
    return out.reshape(_EMB)


# speedup vs baseline: 32.0863x; 1.0710x over previous
import functools

import jax
import jax.numpy as jnp
from jax import lax
from jax.experimental import pallas as pl
from jax.experimental.pallas import tpu as pltpu
from jax.experimental.pallas import tpu_sc as plsc

_N = 10000
_E = 320000
_DIN = 128
_H1 = 512
_H2 = 256
_EMB = 128

_NC = 2
_NS = 16
_NW = _NC * _NS
_C = 125
_NROW = _E // _C
_NCH = _NROW // _NS
_DCH = _NROW // _NW
_NBUF = 4
_RPT = _N // _NS

_mesh = plsc.VectorSubcoreMesh(core_axis_name="c", subcore_axis_name="s")
_sc_params = pltpu.CompilerParams(use_tc_tiling_on_sc=False)


def _deg_body(dst_hbm, zeros_hbm, ones_hbm, out_hbm, idx_v, ones_v, acc_sh,
              *sems):
    c = lax.axis_index("c")
    s = lax.axis_index("s")
    w = s * _NC + c
    pltpu.sync_copy(zeros_hbm.at[pl.ds(s * _RPT, _RPT)],
                    acc_sh.at[pl.ds(s * _RPT, _RPT)])
    pltpu.sync_copy(dst_hbm.at[pl.ds(w * _DCH, _DCH)], idx_v)
    pltpu.sync_copy(ones_hbm, ones_v)
    plsc.subcore_barrier()

    def group(g, carry):
        for b in range(_NBUF):
            k = g * _NBUF + b

            @pl.when(g > 0)
            def _():
                pltpu.make_async_copy(ones_v, acc_sh.at[idx_v.at[k - _NBUF]],
                                      sems[b]).wait()

            pltpu.async_copy(ones_v, acc_sh.at[idx_v.at[k]], sems[b], add=True)
        return carry

    lax.fori_loop(0, _DCH // _NBUF, group, 0)
    for b in range(_NBUF):
        k = _DCH - _NBUF + b
        pltpu.make_async_copy(ones_v, acc_sh.at[idx_v.at[k]], sems[b]).wait()
    plsc.subcore_barrier()
    pltpu.sync_copy(acc_sh.at[pl.ds(s * _RPT, _RPT)],
                    out_hbm.at[c, pl.ds(s * _RPT, _RPT)])


_deg = pl.kernel(
    _deg_body,
    out_type=jax.ShapeDtypeStruct((_NC, _N, 16), jnp.float32),
    mesh=_mesh,
    scratch_types=[
        pltpu.VMEM((_DCH, _C), jnp.int32),
        pltpu.VMEM((_C, 16), jnp.float32),
        pltpu.VMEM_SHARED((_N, 16), jnp.float32),
    ] + [pltpu.SemaphoreType.DMA] * _NBUF,
    compiler_params=_sc_params,
)


@functools.cache
def _agg(dh, nbuf):
    ni = 2 * nbuf

    def body(x_hbm, sd_hbm, out_hbm, idx_v, rows_v, acc_sh, *sems):
        isem = sems[:ni]
        gsem = sems[ni:ni + nbuf]
        ssem = sems[ni + nbuf:]
        c = lax.axis_index("c")
        s = lax.axis_index("s")
        row0 = s * _NCH
        for q in range(ni):
            pltpu.async_copy(sd_hbm.at[c, row0 + q], idx_v.at[q], isem[q])
        pltpu.sync_copy(x_hbm.at[pl.ds(c * _N + s * _RPT, _RPT)],
                        acc_sh.at[pl.ds(s * _RPT, _RPT)])
        for b in range(nbuf):
            pltpu.make_async_copy(sd_hbm.at[c, row0 + b], idx_v.at[b],
                                  isem[b]).wait()
            pltpu.async_copy(x_hbm.at[idx_v.at[b, 0]], rows_v.at[b], gsem[b])
        plsc.subcore_barrier()

        def group(g, carry):
            for p in range(2):
                k0 = g * ni + p * nbuf
                for b in range(nbuf):
                    j = p * nbuf + b
                    pltpu.make_async_copy(x_hbm.at[idx_v.at[j, 0]],
                                          rows_v.at[b], gsem[b]).wait()
                    pltpu.async_copy(rows_v.at[b], acc_sh.at[idx_v.at[j, 1]],
                                     ssem[b], add=True)
                for b in range(nbuf):
                    j = p * nbuf + b
                    jn = (j + nbuf) % ni
                    k = k0 + b
                    pltpu.make_async_copy(rows_v.at[b],
                                          acc_sh.at[idx_v.at[j, 1]],
                                          ssem[b]).wait()

                    @pl.when(k + nbuf < _NCH)
                    def _():
                        pltpu.make_async_copy(sd_hbm.at[c, row0 + k + nbuf],
                                              idx_v.at[jn], isem[jn]).wait()
                        pltpu.async_copy(x_hbm.at[idx_v.at[jn, 0]],
                                         rows_v.at[b], gsem[b])

                    @pl.when(k + ni < _NCH)
                    def _():
                        pltpu.async_copy(sd_hbm.at[c, row0 + k + ni],
                                         idx_v.at[j], isem[j])
            return carry

        lax.fori_loop(0, _NCH // ni, group, 0)
        plsc.subcore_barrier()
        pltpu.sync_copy(acc_sh.at[pl.ds(s * _RPT, _RPT)],
                        out_hbm.at[c, pl.ds(s * _RPT, _RPT)])

    return pl.kernel(
        body,
        out_type=jax.ShapeDtypeStruct((_NC, _N, dh), jnp.float32),
        mesh=_mesh,
        scratch_types=[
            pltpu.VMEM((ni, 2, _C), jnp.int32),
            pltpu.VMEM((nbuf, _C, dh), jnp.float32),
            pltpu.VMEM_SHARED((_N, dh), jnp.float32),
        ] + [pltpu.SemaphoreType.DMA] * (ni + 2 * nbuf),
        compiler_params=_sc_params,
    )


_BLK = 1000
_GRID = _N // _BLK


def _prep_body(feat_ref, degp_ref, x2_ref, dis_ref):
    deg = 1.0 + degp_ref[0, :, 0:1] + degp_ref[1, :, 0:1]
    dis = lax.rsqrt(deg)
    dis_ref[...] = dis
    x2_ref[0] = feat_ref[:, : _DIN // 2] * dis
    x2_ref[1] = feat_ref[:, _DIN // 2:] * dis


def _prep(features, degp):
    return pl.pallas_call(
        _prep_body,
        grid=(_GRID,),
        in_specs=[
            pl.BlockSpec((_BLK, _DIN), lambda i: (i, 0)),
            pl.BlockSpec((_NC, _BLK, 16), lambda i: (0, i, 0)),
        ],
        out_specs=[
            pl.BlockSpec((_NC, _BLK, _DIN // 2), lambda i: (0, i, 0)),
            pl.BlockSpec((_BLK, 1), lambda i: (i, 0)),
        ],
        out_shape=[
            jax.ShapeDtypeStruct((_NC, _N, _DIN // 2), jnp.float32),
            jax.ShapeDtypeStruct((_N, 1), jnp.float32),
        ],
    )(features, degp)


def _mm_t(a, w):
    return lax.dot_general(a, w, (((1,), (1,)), ((), ())),
                           preferred_element_type=jnp.float32)


def _mid_body(t1_ref, dis_ref, W1_ref, b1_ref, W2_ref, x3a_ref, x3b_ref):
    dis = dis_ref[...]
    aggx = jnp.concatenate([t1_ref[0], t1_ref[1]], axis=1) * dis
    u = jnp.maximum(_mm_t(aggx, W1_ref[...]) + b1_ref[...], 0.0)
    v = _mm_t(u, W2_ref[...]) * dis
    q = _H2 // 4
    x3a_ref[0] = v[:, 0 * q:1 * q]
    x3a_ref[1] = v[:, 1 * q:2 * q]
    x3b_ref[0] = v[:, 2 * q:3 * q]
    x3b_ref[1] = v[:, 3 * q:4 * q]


def _mid(t1, dis, W1, b1, W2):
    return pl.pallas_call(
        _mid_body,
        grid=(_GRID,),
        in_specs=[
            pl.BlockSpec((_NC, _BLK, _DIN // 2), lambda i: (0, i, 0)),
            pl.BlockSpec((_BLK, 1), lambda i: (i, 0)),
            pl.BlockSpec((_H1, _DIN), lambda i: (0, 0)),
            pl.BlockSpec((1, _H1), lambda i: (0, 0)),
            pl.BlockSpec((_H2, _H1), lambda i: (0, 0)),
        ],
        out_specs=[
            pl.BlockSpec((_NC, _BLK, _H2 // 4), lambda i: (0, i, 0)),
            pl.BlockSpec((_NC, _BLK, _H2 // 4), lambda i: (0, i, 0)),
        ],
        out_shape=[
            jax.ShapeDtypeStruct((_NC, _N, _H2 // 4), jnp.float32),
            jax.ShapeDtypeStruct((_NC, _N, _H2 // 4), jnp.float32),
        ],
    )(t1, dis, W1, b1, W2)


def _fin_body(t2a_ref, t2b_ref, dis_ref, b2_ref, Wfc_ref, bfc_ref, out_ref,
              gmax_scr):
    i = pl.program_id(0)
    w = jnp.concatenate([t2a_ref[0], t2a_ref[1], t2b_ref[0], t2b_ref[1]],
                        axis=1) * dis_ref[...] + b2_ref[...]
    m = jnp.max(w, axis=0, keepdims=True)

    @pl.when(i == 0)
    def _():
        gmax_scr[...] = m

    @pl.when(i > 0)
    def _():
        gmax_scr[...] = jnp.maximum(gmax_scr[...], m)

    @pl.when(i == pl.num_programs(0) - 1)
    def _():
        out_ref[...] = _mm_t(gmax_scr[...], Wfc_ref[...]) + bfc_ref[...]


def _fin(t2a, t2b, dis, b2, Wfc, bfc):
    return pl.pallas_call(
        _fin_body,
        grid=(_GRID,),
        in_specs=[
            pl.BlockSpec((_NC, _BLK, _H2 // 4), lambda i: (0, i, 0)),
            pl.BlockSpec((_NC, _BLK, _H2 // 4), lambda i: (0, i, 0)),
            pl.BlockSpec((_BLK, 1), lambda i: (i, 0)),
            pl.BlockSpec((1, _H2), lambda i: (0, 0)),
            pl.BlockSpec((_EMB, _H2), lambda i: (0, 0)),
            pl.BlockSpec((1, _EMB), lambda i: (0, 0)),
        ],
        out_specs=pl.BlockSpec((1, _EMB), lambda i: (0, 0)),
        out_shape=jax.ShapeDtypeStruct((1, _EMB), jnp.float32),
        scratch_shapes=[pltpu.VMEM((1, _H2), jnp.float32)],
    )(t2a, t2b, dis, b2, Wfc, bfc)


def kernel(features, edge_index, W1, b1, W2, b2, Wfc, bfc):
    src = edge_index[0].astype(jnp.int32)
    dst = edge_index[1].astype(jnp.int32)
    srcr = src.reshape(1, _NROW, 1, _C)
    dstr = dst.reshape(1, _NROW, 1, _C)
    off = jnp.arange(_NC, dtype=jnp.int32).reshape(_NC, 1, 1, 1) * _N
    sd = jnp.concatenate(
        [jnp.broadcast_to(srcr + off, (_NC, _NROW, 1, _C)),
         jnp.broadcast_to(dstr, (_NC, _NROW, 1, _C))], axis=2)
    dst3 = dst.reshape(_NROW, _C)
    zeros_n16 = jnp.zeros((_N, 16), jnp.float32)
    ones_c16 = jnp.ones((_C, 16), jnp.float32)

    degp = _deg(dst3, zeros_n16, ones_c16)
    x2, dis = _prep(features.astype(jnp.float32), degp)
    t1 = _agg(_DIN // 2, 5)(x2.reshape(2 * _N, _DIN // 2), sd)
    x3a, x3b = _mid(t1, dis, W1, b1.reshape(1, _H1), W2)
    t2a = _agg(_H2 // 4, 5)(x3a.reshape(2 * _N, _H2 // 4), sd)
    t2b = _agg(_H2 // 4, 5)(x3b.reshape(2 * _N, _H2 // 4), sd)
    out = _fin(t2a, t2b, dis, b2.reshape(1, _H2), Wfc, bfc.reshape(1, _EMB))
    return out.reshape(_EMB)


# speedup vs baseline: 32.1494x; 1.0020x over previous
import functools

import jax
import jax.numpy as jnp
from jax import lax
from jax.experimental import pallas as pl
from jax.experimental.pallas import tpu as pltpu
from jax.experimental.pallas import tpu_sc as plsc

_N = 10000
_E = 320000
_DIN = 128
_H1 = 512
_H2 = 256
_EMB = 128

_NC = 2
_NS = 16
_NW = _NC * _NS
_C = 125
_NROW = _E // _C
_NCH = _NROW // _NS
_DCH = _NROW // _NW
_NBUF = 4
_RPT = _N // _NS

_mesh = plsc.VectorSubcoreMesh(core_axis_name="c", subcore_axis_name="s")
_sc_params = pltpu.CompilerParams(use_tc_tiling_on_sc=False)


def _deg_body(dst_hbm, zeros_hbm, ones_hbm, out_hbm, idx_v, ones_v, acc_sh,
              *sems):
    c = lax.axis_index("c")
    s = lax.axis_index("s")
    w = s * _NC + c
    pltpu.sync_copy(zeros_hbm.at[pl.ds(s * _RPT, _RPT)],
                    acc_sh.at[pl.ds(s * _RPT, _RPT)])
    pltpu.sync_copy(dst_hbm.at[pl.ds(w * _DCH, _DCH)], idx_v)
    pltpu.sync_copy(ones_hbm, ones_v)
    plsc.subcore_barrier()

    def group(g, carry):
        for b in range(_NBUF):
            k = g * _NBUF + b

            @pl.when(g > 0)
            def _():
                pltpu.make_async_copy(ones_v, acc_sh.at[idx_v.at[k - _NBUF]],
                                      sems[b]).wait()

            pltpu.async_copy(ones_v, acc_sh.at[idx_v.at[k]], sems[b], add=True)
        return carry

    lax.fori_loop(0, _DCH // _NBUF, group, 0)
    for b in range(_NBUF):
        k = _DCH - _NBUF + b
        pltpu.make_async_copy(ones_v, acc_sh.at[idx_v.at[k]], sems[b]).wait()
    plsc.subcore_barrier()
    pltpu.sync_copy(acc_sh.at[pl.ds(s * _RPT, _RPT)],
                    out_hbm.at[c, pl.ds(s * _RPT, _RPT)])


_deg = pl.kernel(
    _deg_body,
    out_type=jax.ShapeDtypeStruct((_NC, _N, 16), jnp.float32),
    mesh=_mesh,
    scratch_types=[
        pltpu.VMEM((_DCH, _C), jnp.int32),
        pltpu.VMEM((_C, 16), jnp.float32),
        pltpu.VMEM_SHARED((_N, 16), jnp.float32),
    ] + [pltpu.SemaphoreType.DMA] * _NBUF,
    compiler_params=_sc_params,
)


@functools.cache
def _agg(dh, nbuf):
    ni = 2 * nbuf

    def body(x_hbm, sd_hbm, out_hbm, idx_v, rows_v, acc_sh, *sems):
        isem = sems[:ni]
        gsem = sems[ni:ni + nbuf]
        ssem = sems[ni + nbuf:]
        c = lax.axis_index("c")
        s = lax.axis_index("s")
        row0 = s * _NCH
        for q in range(ni):
            pltpu.async_copy(sd_hbm.at[c, row0 + q], idx_v.at[q], isem[q])
        pltpu.sync_copy(x_hbm.at[pl.ds(c * _N + s * _RPT, _RPT)],
                        acc_sh.at[pl.ds(s * _RPT, _RPT)])
        for b in range(nbuf):
            pltpu.make_async_copy(sd_hbm.at[c, row0 + b], idx_v.at[b],
                                  isem[b]).wait()
            pltpu.async_copy(x_hbm.at[idx_v.at[b, 0]], rows_v.at[b], gsem[b])
        plsc.subcore_barrier()

        def group(g, carry):
            for p in range(2):
                k0 = g * ni + p * nbuf
                for b in range(nbuf):
                    j = p * nbuf + b
                    pltpu.make_async_copy(x_hbm.at[idx_v.at[j, 0]],
                                          rows_v.at[b], gsem[b]).wait()
                    pltpu.async_copy(rows_v.at[b], acc_sh.at[idx_v.at[j, 1]],
                                     ssem[b], add=True)
                for b in range(nbuf):
                    j = p * nbuf + b
                    jn = (j + nbuf) % ni
                    k = k0 + b
                    pltpu.make_async_copy(rows_v.at[b],
                                          acc_sh.at[idx_v.at[j, 1]],
                                          ssem[b]).wait()

                    @pl.when(k + nbuf < _NCH)
                    def _():
                        pltpu.make_async_copy(sd_hbm.at[c, row0 + k + nbuf],
                                              idx_v.at[jn], isem[jn]).wait()
                        pltpu.async_copy(x_hbm.at[idx_v.at[jn, 0]],
                                         rows_v.at[b], gsem[b])

                    @pl.when(k + ni < _NCH)
                    def _():
                        pltpu.async_copy(sd_hbm.at[c, row0 + k + ni],
                                         idx_v.at[j], isem[j])
            return carry

        lax.fori_loop(0, _NCH // ni, group, 0)
        plsc.subcore_barrier()
        pltpu.sync_copy(acc_sh.at[pl.ds(s * _RPT, _RPT)],
                        out_hbm.at[c, pl.ds(s * _RPT, _RPT)])

    return pl.kernel(
        body,
        out_type=jax.ShapeDtypeStruct((_NC, _N, dh), jnp.float32),
        mesh=_mesh,
        scratch_types=[
            pltpu.VMEM((ni, 2, _C), jnp.int32),
            pltpu.VMEM((nbuf, _C, dh), jnp.float32),
            pltpu.VMEM_SHARED((_N, dh), jnp.float32),
        ] + [pltpu.SemaphoreType.DMA] * (ni + 2 * nbuf),
        compiler_params=_sc_params,
    )


_BLK = 1000
_GRID = _N // _BLK


def _prep_body(feat_ref, degp_ref, x2_ref, dis_ref):
    deg = 1.0 + degp_ref[0, :, 0:1] + degp_ref[1, :, 0:1]
    dis = lax.rsqrt(deg)
    dis_ref[...] = dis
    x2_ref[0] = feat_ref[:, : _DIN // 2] * dis
    x2_ref[1] = feat_ref[:, _DIN // 2:] * dis


def _prep(features, degp):
    return pl.pallas_call(
        _prep_body,
        grid=(_GRID,),
        in_specs=[
            pl.BlockSpec((_BLK, _DIN), lambda i: (i, 0)),
            pl.BlockSpec((_NC, _BLK, 16), lambda i: (0, i, 0)),
        ],
        out_specs=[
            pl.BlockSpec((_NC, _BLK, _DIN // 2), lambda i: (0, i, 0)),
            pl.BlockSpec((_BLK, 1), lambda i: (i, 0)),
        ],
        out_shape=[
            jax.ShapeDtypeStruct((_NC, _N, _DIN // 2), jnp.float32),
            jax.ShapeDtypeStruct((_N, 1), jnp.float32),
        ],
    )(features, degp)


def _mm_t(a, w):
    return lax.dot_general(a, w, (((1,), (1,)), ((), ())),
                           preferred_element_type=jnp.float32)


def _mid_body(t1_ref, dis_ref, W1_ref, b1_ref, W2_ref, x3a_ref, x3b_ref):
    dis = dis_ref[...]
    aggx = jnp.concatenate([t1_ref[0], t1_ref[1]],
                           axis=1).astype(jnp.float32) * dis
    u = jnp.maximum(_mm_t(aggx, W1_ref[...]) + b1_ref[...], 0.0)
    v = _mm_t(u, W2_ref[...]) * dis
    q = _H2 // 4
    x3a_ref[0] = v[:, 0 * q:1 * q]
    x3a_ref[1] = v[:, 1 * q:2 * q]
    x3b_ref[0] = v[:, 2 * q:3 * q]
    x3b_ref[1] = v[:, 3 * q:4 * q]


def _mid(t1, dis, W1, b1, W2):
    return pl.pallas_call(
        _mid_body,
        grid=(_GRID,),
        in_specs=[
            pl.BlockSpec((_NC, _BLK, _DIN // 2), lambda i: (0, i, 0)),
            pl.BlockSpec((_BLK, 1), lambda i: (i, 0)),
            pl.BlockSpec((_H1, _DIN), lambda i: (0, 0)),
            pl.BlockSpec((1, _H1), lambda i: (0, 0)),
            pl.BlockSpec((_H2, _H1), lambda i: (0, 0)),
        ],
        out_specs=[
            pl.BlockSpec((_NC, _BLK, _H2 // 4), lambda i: (0, i, 0)),
            pl.BlockSpec((_NC, _BLK, _H2 // 4), lambda i: (0, i, 0)),
        ],
        out_shape=[
            jax.ShapeDtypeStruct((_NC, _N, _H2 // 4), jnp.float32),
            jax.ShapeDtypeStruct((_NC, _N, _H2 // 4), jnp.float32),
        ],
    )(t1, dis, W1, b1, W2)


def _fin_body(t2a_ref, t2b_ref, dis_ref, b2_ref, Wfc_ref, bfc_ref, out_ref,
              gmax_scr):
    i = pl.program_id(0)
    w = jnp.concatenate([t2a_ref[0], t2a_ref[1], t2b_ref[0], t2b_ref[1]],
                        axis=1).astype(jnp.float32) * dis_ref[...] + b2_ref[...]
    m = jnp.max(w, axis=0, keepdims=True)

    @pl.when(i == 0)
    def _():
        gmax_scr[...] = m

    @pl.when(i > 0)
    def _():
        gmax_scr[...] = jnp.maximum(gmax_scr[...], m)

    @pl.when(i == pl.num_programs(0) - 1)
    def _():
        out_ref[...] = _mm_t(gmax_scr[...], Wfc_ref[...]) + bfc_ref[...]


def _fin(t2a, t2b, dis, b2, Wfc, bfc):
    return pl.pallas_call(
        _fin_body,
        grid=(_GRID,),
        in_specs=[
            pl.BlockSpec((_NC, _BLK, _H2 // 4), lambda i: (0, i, 0)),
            pl.BlockSpec((_NC, _BLK, _H2 // 4), lambda i: (0, i, 0)),
            pl.BlockSpec((_BLK, 1), lambda i: (i, 0)),
            pl.BlockSpec((1, _H2), lambda i: (0, 0)),
            pl.BlockSpec((_EMB, _H2), lambda i: (0, 0)),
            pl.BlockSpec((1, _EMB), lambda i: (0, 0)),
        ],
        out_specs=pl.BlockSpec((1, _EMB), lambda i: (0, 0)),
        out_shape=jax.ShapeDtypeStruct((1, _EMB), jnp.float32),
        scratch_shapes=[pltpu.VMEM((1, _H2), jnp.float32)],
    )(t2a, t2b, dis, b2, Wfc, bfc)


def kernel(features, edge_index, W1, b1, W2, b2, Wfc, bfc):
    src = edge_index[0].astype(jnp.int32)
    dst = edge_index[1].astype(jnp.int32)
    srcr = src.reshape(1, _NROW, 1, _C)
    dstr = dst.reshape(1, _NROW, 1, _C)
    off = jnp.arange(_NC, dtype=jnp.int32).reshape(_NC, 1, 1, 1) * _N
    sd = jnp.concatenate(
        [jnp.broadcast_to(srcr + off, (_NC, _NROW, 1, _C)),
         jnp.broadcast_to(dstr, (_NC, _NROW, 1, _C))], axis=2)
    dst3 = dst.reshape(_NROW, _C)
    zeros_n16 = jnp.zeros((_N, 16), jnp.float32)
    ones_c16 = jnp.ones((_C, 16), jnp.float32)

    degp = _deg(dst3, zeros_n16, ones_c16)
    x2, dis = _prep(features.astype(jnp.float32), degp)
    t1 = _agg(_DIN // 2, 5)(x2.reshape(2 * _N, _DIN // 2), sd)
    x3a, x3b = _mid(t1, dis, W1, b1.reshape(1, _H1), W2)
    t2a = _agg(_H2 // 4, 5)(x3a.reshape(2 * _N, _H2 // 4), sd)
    t2b = _agg(_H2 // 4, 5)(x3b.reshape(2 * _N, _H2 // 4), sd)
    out = _fin(t2a, t2b, dis, b2.reshape(1, _H2), Wfc, bfc.reshape(1, _EMB))
    return out.reshape(_EMB)
